# baseline (speedup 1.0000x reference)
import functools

import jax
import jax.numpy as jnp
from jax import lax
from jax.experimental import pallas as pl
from jax.experimental.pallas import tpu as pltpu
from jax.experimental.pallas import tpu_sc as plsc

_LANES = 16
_NBINS = 64
_BIAS = 24
_SHRINK = 1.0 - 2.0 ** -23
_MAGIC = float(2 ** 23)


def _make_sc_call(n, nw, chunk, unroll):
    per_w = n // nw
    n_chunks = per_w // chunk
    n_pairs = n_chunks // 2
    mesh = plsc.VectorSubcoreMesh(core_axis_name="c", subcore_axis_name="s")

    @functools.partial(
        pl.kernel,
        mesh=mesh,
        out_type=jax.ShapeDtypeStruct((nw, _LANES), jnp.float32),
        compiler_params=pltpu.CompilerParams(needs_layout_passes=False),
        scratch_types=[
            pltpu.VMEM((chunk,), jnp.float32),
            pltpu.VMEM((chunk,), jnp.float32),
            pltpu.VMEM((chunk,), jnp.float32),
            pltpu.VMEM((chunk,), jnp.float32),
            pltpu.VMEM((_NBINS * _LANES,), jnp.float32),
            pltpu.VMEM((_LANES,), jnp.float32),
            pltpu.VMEM((_LANES,), jnp.float32),
            pltpu.VMEM((_LANES,), jnp.float32),
            pltpu.SemaphoreType.DMA,
            pltpu.SemaphoreType.DMA,
        ],
    )
    def run(pred_hbm, target_hbm, edges_hbm, weights_hbm, out_hbm,
            pbuf0, pbuf1, tbuf0, tbuf1, bins, wv, ev, accv, sem0, sem1):
        pbufs = (pbuf0, pbuf1)
        tbufs = (tbuf0, tbuf1)
        sems = (sem0, sem1)
        cid = lax.axis_index("c")
        sid = lax.axis_index("s")
        wid = sid * 2 + cid
        shard = wid * per_w

        def start(ci, b):
            src = pl.ds(shard + ci * chunk, chunk)
            pltpu.make_async_copy(pred_hbm.at[src], pbufs[b], sems[b]).start()
            pltpu.make_async_copy(target_hbm.at[src], tbufs[b], sems[b]).start()

        def wait(b):
            drain = pl.ds(0, chunk)
            pltpu.make_async_copy(pred_hbm.at[drain], pbufs[b], sems[b]).wait()
            pltpu.make_async_copy(target_hbm.at[drain], tbufs[b], sems[b]).wait()

        start(0, 0)
        start(1, 1)
        pltpu.sync_copy(weights_hbm, wv)
        pltpu.sync_copy(edges_hbm.at[pl.ds(0, _LANES)], ev)

        zero = jnp.zeros((_LANES,), jnp.float32)

        def zero_body(k, c):
            bins[pl.ds(k * _LANES, _LANES)] = zero
            return c
        lax.fori_loop(0, _NBINS, zero_body, 0)

        evec = ev[...]
        b0 = jnp.full((_LANES,), evec[0], jnp.float32)
        b1 = jnp.full((_LANES,), evec[1], jnp.float32)
        vscale = _SHRINK / (b1 - b0)
        voff = (-b0) * vscale + (_BIAS - 0.5)
        scale = vscale[0]
        off = voff[0]
        lane = lax.iota(jnp.int32, _LANES)

        step = _LANES * unroll

        def make_body(pref, tref):
            def body(vi, c):
                base = vi * step
                for u in range(unroll):
                    s = pl.ds(base + u * _LANES, _LANES)
                    t = tref[s]
                    p = pref[s]
                    d = p - t
                    y = (t * scale + off) + _MAGIC
                    k = plsc.bitcast(y, jnp.int32) & (_NBINS - 1)
                    addr = lax.shift_left(k, 4) + lane
                    plsc.addupdate_scatter(bins, [addr], d * d)
                return c
            return body

        def compute(b):
            lax.fori_loop(0, chunk // step, make_body(pbufs[b], tbufs[b]), 0)

        def pair_body(cp, c):
            ci0 = cp * 2
            wait(0)
            compute(0)
            start(ci0 + 2, 0)
            wait(1)
            compute(1)
            start(ci0 + 3, 1)
            return c
        lax.fori_loop(0, n_pairs - 1, pair_body, 0)
        wait(0)
        compute(0)
        wait(1)
        compute(1)

        acc = zero
        for g in range(4):
            widx = lane + (g * _LANES - _BIAS)
            widx = jnp.minimum(jnp.maximum(widx, 0), _LANES - 1)
            wg = plsc.load_gather(wv, [widx])
            for j in range(_LANES):
                col = bins[pl.ds((g * _LANES + j) * _LANES, _LANES)]
                acc = acc + col * wg[j]
        accv[...] = acc
        pltpu.sync_copy(accv, out_hbm.at[wid])

    return run


def kernel(pred, target, bin_edges, weights):
    n = pred.shape[0]
    info = plsc.get_sparse_core_info()
    nw = info.num_cores * info.num_subcores
    run = _make_sc_call(n, nw, chunk=16384, unroll=8)
    partials = run(pred, target, bin_edges, weights)
    return jnp.sum(partials) / n


# speedup vs baseline: 2.8379x; 2.8379x over previous
import functools

import jax
import jax.numpy as jnp
from jax import lax
from jax.experimental import pallas as pl
from jax.experimental.pallas import tpu as pltpu
from jax.experimental.pallas import tpu_sc as plsc

_LANES = 16
_NBINS = 64
_BIAS = 24
_SHRINK = 1.0 - 2.0 ** -23
_MAGIC = float(2 ** 23)


def _make_sc_call(n, nw, chunk, unroll):
    per_w = n // nw
    n_chunks = per_w // chunk
    n_pairs = n_chunks // 2
    mesh = plsc.VectorSubcoreMesh(core_axis_name="c", subcore_axis_name="s")

    @functools.partial(
        pl.kernel,
        mesh=mesh,
        out_type=jax.ShapeDtypeStruct((nw, _LANES), jnp.float32),
        compiler_params=pltpu.CompilerParams(needs_layout_passes=False),
        scratch_types=[
            pltpu.VMEM((chunk,), jnp.float32),
            pltpu.VMEM((chunk,), jnp.float32),
            pltpu.VMEM((chunk,), jnp.float32),
            pltpu.VMEM((chunk,), jnp.float32),
            pltpu.VMEM((_NBINS,), jnp.float32),
            pltpu.VMEM((_LANES,), jnp.float32),
            pltpu.VMEM((_LANES,), jnp.float32),
            pltpu.VMEM((_LANES,), jnp.float32),
            pltpu.SemaphoreType.DMA,
            pltpu.SemaphoreType.DMA,
        ],
    )
    def run(pred_hbm, target_hbm, edges_hbm, weights_hbm, out_hbm,
            pbuf0, pbuf1, tbuf0, tbuf1, w64, wv, ev, accv, sem0, sem1):
        pbufs = (pbuf0, pbuf1)
        tbufs = (tbuf0, tbuf1)
        sems = (sem0, sem1)
        cid = lax.axis_index("c")
        sid = lax.axis_index("s")
        wid = sid * 2 + cid
        shard = wid * per_w

        def start(ci, b):
            src = pl.ds(shard + ci * chunk, chunk)
            pltpu.make_async_copy(pred_hbm.at[src], pbufs[b], sems[b]).start()
            pltpu.make_async_copy(target_hbm.at[src], tbufs[b], sems[b]).start()

        def wait(b):
            drain = pl.ds(0, chunk)
            pltpu.make_async_copy(pred_hbm.at[drain], pbufs[b], sems[b]).wait()
            pltpu.make_async_copy(target_hbm.at[drain], tbufs[b], sems[b]).wait()

        start(0, 0)
        start(1, 1)
        pltpu.sync_copy(weights_hbm, wv)
        pltpu.sync_copy(edges_hbm.at[pl.ds(0, _LANES)], ev)

        lane = lax.iota(jnp.int32, _LANES)
        for g in range(_NBINS // _LANES):
            widx = lane + (g * _LANES - _BIAS)
            widx = jnp.minimum(jnp.maximum(widx, 0), _LANES - 1)
            w64[pl.ds(g * _LANES, _LANES)] = plsc.load_gather(wv, [widx])

        evec = ev[...]
        b0 = jnp.full((_LANES,), evec[0], jnp.float32)
        b1 = jnp.full((_LANES,), evec[1], jnp.float32)
        vscale = _SHRINK / (b1 - b0)
        voff = (-b0) * vscale + (_BIAS - 0.5)
        scale = vscale[0]
        off = voff[0]

        n_acc = 4
        step = _LANES * unroll

        def make_body(pref, tref):
            def body(vi, accs):
                accs = list(accs)
                base = vi * step
                for u in range(unroll):
                    s = pl.ds(base + u * _LANES, _LANES)
                    t = tref[s]
                    p = pref[s]
                    d = p - t
                    y = (t * scale + off) + _MAGIC
                    k = plsc.bitcast(y, jnp.int32) & (_NBINS - 1)
                    w = plsc.load_gather(w64, [k])
                    accs[u % n_acc] = accs[u % n_acc] + (d * d) * w
                return tuple(accs)
            return body

        def compute(b, accs):
            return lax.fori_loop(0, chunk // step,
                                 make_body(pbufs[b], tbufs[b]), accs)

        accs = tuple(jnp.zeros((_LANES,), jnp.float32) for _ in range(n_acc))

        def pair_body(cp, accs):
            ci0 = cp * 2
            wait(0)
            accs = compute(0, accs)
            start(ci0 + 2, 0)
            wait(1)
            accs = compute(1, accs)
            start(ci0 + 3, 1)
            return accs
        accs = lax.fori_loop(0, n_pairs - 1, pair_body, accs)
        wait(0)
        accs = compute(0, accs)
        wait(1)
        accs = compute(1, accs)

        acc = (accs[0] + accs[1]) + (accs[2] + accs[3])
        accv[...] = acc
        pltpu.sync_copy(accv, out_hbm.at[wid])

    return run


def kernel(pred, target, bin_edges, weights):
    n = pred.shape[0]
    info = plsc.get_sparse_core_info()
    nw = info.num_cores * info.num_subcores
    run = _make_sc_call(n, nw, chunk=16384, unroll=8)
    partials = run(pred, target, bin_edges, weights)
    return jnp.sum(partials) / n
